# Initial kernel scaffold; baseline (speedup 1.0000x reference)
import jax
import jax.numpy as jnp
from jax.experimental import pallas as pl


def kernel(x, params):
    raise NotImplementedError("write your pallas kernel here")


# speedup vs baseline: 3.5909x; 3.5909x over previous
import functools

import jax
import jax.numpy as jnp
from jax import lax
from jax.experimental import pallas as pl
from jax.experimental.pallas import tpu as pltpu
from jax.experimental.pallas import tpu_sc as plsc

_B, _T, _D = 4, 8192, 1024
_NC, _NS = 2, 16
_NW = _NC * _NS
_RPW = _T // _NW
_CH = 64
_NCHUNK = _RPW // _CH


def _make_sc_broadcast():
  mesh = plsc.VectorSubcoreMesh(core_axis_name="c", subcore_axis_name="s")

  @functools.partial(
      pl.kernel,
      out_type=jax.ShapeDtypeStruct((_B, _T, _D), jnp.float32),
      mesh=mesh,
      scratch_types=[
          pltpu.VMEM((_CH, _D), jnp.float32),
          pltpu.SemaphoreType.DMA,
      ],
  )
  def body(params_hbm, out_hbm, buf, sem):
    wid = lax.axis_index("s") * _NC + lax.axis_index("c")
    for k in range(_NCHUNK):
      base = wid * _RPW + k * _CH
      pltpu.sync_copy(params_hbm.at[pl.ds(base, _CH)], buf)
      copies = [
          pltpu.async_copy(buf, out_hbm.at[n, pl.ds(base, _CH)], sem)
          for n in range(_B)
      ]
      for cp in copies:
        cp.wait()

  return body


_sc_broadcast = _make_sc_broadcast()


@jax.jit
def kernel(x, params):
  del x
  return _sc_broadcast(params)
